# baseline (speedup 1.0000x reference)
import functools

import jax
import jax.numpy as jnp
from jax import lax
from jax.experimental import pallas as pl
from jax.experimental.pallas import tpu as pltpu
from jax.experimental.pallas import tpu_sc as plsc

N = 100000
E = 6400000
LAMBDA_SMOOTH = 0.1
LAMBDA_GEO = 0.1
LAMBDA_GRADIENT = 0.05

NC = 2
NS = 16
NW = NC * NS
CHUNK = 2048
VPG = CHUNK // 16
TOTAL_CHUNKS = E // CHUNK
FULL_ROUNDS = 96


def _rsqrt16(x):
    i = lax.bitcast_convert_type(x, jnp.int32)
    i = jnp.int32(0x5F3759DF) - lax.shift_right_arithmetic(i, 1)
    y = lax.bitcast_convert_type(i, jnp.float32)
    y = y * (1.5 - (x * 0.5) * y * y)
    return y


@functools.partial(
    pl.kernel,
    out_type=[
        jax.ShapeDtypeStruct((NW, 16), jnp.float32),
        jax.ShapeDtypeStruct((NW, 16), jnp.float32),
    ],
    mesh=plsc.VectorSubcoreMesh(core_axis_name="c", subcore_axis_name="s"),
    compiler_params=pltpu.CompilerParams(
        needs_layout_passes=False, use_tc_tiling_on_sc=False
    ),
    scratch_types=[
        pltpu.VMEM((2 * CHUNK,), jnp.int32),
        pltpu.VMEM((2 * CHUNK, 8), jnp.float32),
        pltpu.VMEM((2 * CHUNK,), jnp.int32),
        pltpu.VMEM((2 * CHUNK, 8), jnp.float32),
        pltpu.VMEM((16,), jnp.float32),
        pltpu.VMEM((16,), jnp.float32),
        pltpu.VMEM_SHARED((N, 8), jnp.float32),
        pltpu.SemaphoreType.DMA,
        pltpu.SemaphoreType.DMA,
        pltpu.SemaphoreType.DMA,
        pltpu.SemaphoreType.DMA,
    ],
)
def _edge_loss_sc(
    table, edges, out_s, out_g,
    midx0, mrows0, midx1, mrows1,
    accs_v, accg_v, stable, semi0, semi1, semg0, semg1,
):
    src_i = edges.at[0]
    dst_i = edges.at[1]
    sid = lax.axis_index("s")
    wid = lax.axis_index("s") * NC + lax.axis_index("c")

    @pl.when(sid == 0)
    def _():
        pltpu.sync_copy(table, stable)

    plsc.subcore_barrier()
    iota = lax.iota(jnp.int32, 16)
    cols = [jnp.full((16,), c, jnp.int32) for c in range(5)]
    zero = jnp.zeros((16,), jnp.float32)

    bufs = (
        (midx0, mrows0, semi0, semg0),
        (midx1, mrows1, semi1, semg1),
    )

    def idx_start(j, b):
        midx, _, semi, _ = bufs[b]
        c = lax.min(wid + NW * j, TOTAL_CHUNKS - 1)
        base = pl.multiple_of(c * CHUNK, CHUNK)
        pltpu.async_copy(src_i.at[pl.ds(base, CHUNK)], midx.at[pl.ds(0, CHUNK)], semi)
        pltpu.async_copy(dst_i.at[pl.ds(base, CHUNK)], midx.at[pl.ds(CHUNK, CHUNK)], semi)

    def idx_wait(b):
        midx, _, semi, _ = bufs[b]
        pltpu.make_async_copy(src_i.at[pl.ds(0, 2 * CHUNK)], midx, semi).wait()

    def gather_start(b):
        midx, mrows, _, semg = bufs[b]
        pltpu.async_copy(stable.at[midx], mrows, semg)

    def gather_wait(b):
        midx, mrows, _, semg = bufs[b]
        pltpu.make_async_copy(stable.at[midx], mrows, semg).wait()

    def compute(b, sm0, gr0):
        _, mrows, _, _ = bufs[b]

        def vec_body(i, accs2):
            sm, gr = accs2
            ri = i * 16 + iota
            di = ri + CHUNK
            s0 = plsc.load_gather(mrows, [ri, cols[0]])
            t0 = plsc.load_gather(mrows, [di, cols[0]])
            s1 = plsc.load_gather(mrows, [ri, cols[1]])
            t1 = plsc.load_gather(mrows, [di, cols[1]])
            s2 = plsc.load_gather(mrows, [ri, cols[2]])
            t2 = plsc.load_gather(mrows, [di, cols[2]])
            sx = plsc.load_gather(mrows, [ri, cols[3]])
            tx = plsc.load_gather(mrows, [di, cols[3]])
            sy = plsc.load_gather(mrows, [ri, cols[4]])
            ty = plsc.load_gather(mrows, [di, cols[4]])
            d0 = s0 - t0
            d1 = s1 - t1
            d2 = s2 - t2
            sm = sm + (d0 * d0 + (d1 * d1 + d2 * d2))
            dx = sx - tx
            dy = sy - ty
            h2 = jnp.maximum(dx * dx + dy * dy, 1e-12)
            inv = _rsqrt16(h2)
            g0 = jnp.maximum(jnp.abs(d0) * inv - 0.1, 0.0)
            g1 = jnp.maximum(jnp.abs(d1) * inv - 0.1, 0.0)
            g2 = jnp.maximum(jnp.abs(d2) * inv - 0.1, 0.0)
            gr = gr + (g0 + (g1 + g2))
            return sm, gr

        return lax.fori_loop(0, VPG, vec_body, (sm0, gr0), unroll=8)

    idx_start(0, 0)
    idx_wait(0)
    gather_start(0)
    idx_start(1, 1)

    def pair_body(j2, accs):
        sm, gr = accs
        jA = 2 * j2
        gather_wait(0)
        idx_wait(1)
        gather_start(1)
        idx_start(jA + 2, 0)
        sm, gr = compute(0, sm, gr)
        gather_wait(1)
        idx_wait(0)
        gather_start(0)
        idx_start(jA + 3, 1)
        sm, gr = compute(1, sm, gr)
        return sm, gr

    sm, gr = lax.fori_loop(0, FULL_ROUNDS // 2, pair_body, (zero, zero))

    gather_wait(0)
    idx_wait(1)
    gather_start(1)
    sm, gr = compute(0, sm, gr)
    gather_wait(1)
    sm2, gr2 = compute(1, sm, gr)
    valid = (wid + NW * (FULL_ROUNDS + 1)) < TOTAL_CHUNKS
    sm = jnp.where(valid, sm2, sm)
    gr = jnp.where(valid, gr2, gr)

    accs_v[...] = sm
    accg_v[...] = gr
    pltpu.sync_copy(accs_v, out_s.at[wid])
    pltpu.sync_copy(accg_v, out_g.at[wid])


def _combine_tc(pT_ref, tT_ref, ps_ref, pg_ref, out_ref):
    p = pT_ref[...]
    t = tT_ref[...]
    diff = p - t
    recon = jnp.sum(diff * diff) * (1.0 / (3.0 * N))
    th = p[0, :]
    fl = p[1, :]
    ro = p[2, :]
    geo = (
        jnp.sum(jnp.maximum(-th, 0.0))
        + jnp.sum(jnp.maximum(fl - ro + 0.1, 0.0))
        + jnp.sum((th - (ro - fl)) ** 2)
        + jnp.sum(jnp.maximum(th - 20.0, 0.0))
    ) * (1.0 / N)
    smooth = jnp.sum(ps_ref[...]) * (1.0 / (3.0 * E))
    grad = jnp.sum(pg_ref[...]) * (1.0 / (3.0 * E))
    total = recon + LAMBDA_SMOOTH * smooth + LAMBDA_GEO * geo + LAMBDA_GRADIENT * grad
    out_ref[...] = jnp.broadcast_to(total, (1, 1))


def kernel(pred, target, edge_index, positions):
    table = jnp.concatenate(
        [pred, positions[:, :2], jnp.zeros((N, 3), jnp.float32)], axis=1
    )
    part_s, part_g = _edge_loss_sc(table, edge_index)
    out = pl.pallas_call(
        _combine_tc,
        out_shape=jax.ShapeDtypeStruct((1, 1), jnp.float32),
    )(pred.T, target.T, part_s, part_g)
    return out[0, 0]


# speedup vs baseline: 1.4840x; 1.4840x over previous
import functools

import jax
import jax.numpy as jnp
from jax import lax
from jax.experimental import pallas as pl
from jax.experimental.pallas import tpu as pltpu
from jax.experimental.pallas import tpu_sc as plsc

N = 100000
E = 6400000
LAMBDA_SMOOTH = 0.1
LAMBDA_GEO = 0.1
LAMBDA_GRADIENT = 0.05

NC = 2
NS = 16
NW = NC * NS
CHUNK = 2048
VPG = CHUNK // 16
TOTAL_CHUNKS = E // CHUNK
FULL_ROUNDS = 96


def _rsqrt16(x):
    i = lax.bitcast_convert_type(x, jnp.int32)
    i = jnp.int32(0x5F3759DF) - lax.shift_right_arithmetic(i, 1)
    y = lax.bitcast_convert_type(i, jnp.float32)
    y = y * (1.5 - (x * 0.5) * y * y)
    return y


@functools.partial(
    pl.kernel,
    out_type=[
        jax.ShapeDtypeStruct((NW, 16), jnp.float32),
        jax.ShapeDtypeStruct((NW, 16), jnp.float32),
    ],
    mesh=plsc.VectorSubcoreMesh(core_axis_name="c", subcore_axis_name="s"),
    compiler_params=pltpu.CompilerParams(
        needs_layout_passes=False, use_tc_tiling_on_sc=False
    ),
    scratch_types=[
        pltpu.VMEM((2 * CHUNK,), jnp.int32),
        pltpu.VMEM((2 * CHUNK, 8), jnp.float32),
        pltpu.VMEM((2 * CHUNK,), jnp.int32),
        pltpu.VMEM((2 * CHUNK, 8), jnp.float32),
        pltpu.VMEM((16,), jnp.float32),
        pltpu.VMEM((16,), jnp.float32),
        pltpu.VMEM_SHARED((N, 8), jnp.float32),
        pltpu.SemaphoreType.DMA,
        pltpu.SemaphoreType.DMA,
        pltpu.SemaphoreType.DMA,
        pltpu.SemaphoreType.DMA,
    ],
)
def _edge_loss_sc(
    table, edges, out_s, out_g,
    midx0, mrows0, midx1, mrows1,
    accs_v, accg_v, stable, semi0, semi1, semg0, semg1,
):
    src_i = edges.at[0]
    dst_i = edges.at[1]
    sid = lax.axis_index("s")
    wid = lax.axis_index("s") * NC + lax.axis_index("c")

    @pl.when(sid == 0)
    def _():
        pltpu.sync_copy(table, stable)

    plsc.subcore_barrier()
    iota = lax.iota(jnp.int32, 16)
    cols = [jnp.full((16,), c, jnp.int32) for c in range(5)]
    zero = jnp.zeros((16,), jnp.float32)

    bufs = (
        (midx0, mrows0, semi0, semg0),
        (midx1, mrows1, semi1, semg1),
    )

    def idx_start(j, b):
        midx, _, semi, _ = bufs[b]
        c = lax.min(wid + NW * j, TOTAL_CHUNKS - 1)
        base = pl.multiple_of(c * CHUNK, CHUNK)
        pltpu.async_copy(src_i.at[pl.ds(base, CHUNK)], midx.at[pl.ds(0, CHUNK)], semi)
        pltpu.async_copy(dst_i.at[pl.ds(base, CHUNK)], midx.at[pl.ds(CHUNK, CHUNK)], semi)

    def idx_wait(b):
        midx, _, semi, _ = bufs[b]
        pltpu.make_async_copy(src_i.at[pl.ds(0, 2 * CHUNK)], midx, semi).wait()

    def gather_start(b):
        midx, mrows, _, semg = bufs[b]
        pltpu.async_copy(stable.at[midx], mrows, semg)

    def gather_wait(b):
        midx, mrows, _, semg = bufs[b]
        pltpu.make_async_copy(stable.at[midx], mrows, semg).wait()

    def compute(b, sm0, gr0):
        _, mrows, _, _ = bufs[b]

        def vec_body(i, accs2):
            sm, gr = accs2
            ri = i * 16 + iota
            di = ri + CHUNK
            s0 = plsc.load_gather(mrows, [ri, cols[0]])
            t0 = plsc.load_gather(mrows, [di, cols[0]])
            s1 = plsc.load_gather(mrows, [ri, cols[1]])
            t1 = plsc.load_gather(mrows, [di, cols[1]])
            s2 = plsc.load_gather(mrows, [ri, cols[2]])
            t2 = plsc.load_gather(mrows, [di, cols[2]])
            sx = plsc.load_gather(mrows, [ri, cols[3]])
            tx = plsc.load_gather(mrows, [di, cols[3]])
            sy = plsc.load_gather(mrows, [ri, cols[4]])
            ty = plsc.load_gather(mrows, [di, cols[4]])
            d0 = s0 - t0
            d1 = s1 - t1
            d2 = s2 - t2
            sm = sm + (d0 * d0 + (d1 * d1 + d2 * d2))
            dx = sx - tx
            dy = sy - ty
            h2 = jnp.maximum(dx * dx + dy * dy, 1e-12)
            inv = _rsqrt16(h2)
            g0 = jnp.maximum(jnp.abs(d0) * inv - 0.1, 0.0)
            g1 = jnp.maximum(jnp.abs(d1) * inv - 0.1, 0.0)
            g2 = jnp.maximum(jnp.abs(d2) * inv - 0.1, 0.0)
            gr = gr + (g0 + (g1 + g2))
            return sm, gr

        return lax.fori_loop(0, VPG, vec_body, (sm0, gr0), unroll=4)

    idx_start(0, 0)
    idx_wait(0)
    gather_start(0)
    idx_start(1, 1)

    def pair_body(j2, accs):
        sm, gr = accs
        jA = 2 * j2
        gather_wait(0)
        idx_wait(1)
        gather_start(1)
        idx_start(jA + 2, 0)
        sm, gr = compute(0, sm, gr)
        gather_wait(1)
        idx_wait(0)
        gather_start(0)
        idx_start(jA + 3, 1)
        sm, gr = compute(1, sm, gr)
        return sm, gr

    sm, gr = lax.fori_loop(0, FULL_ROUNDS // 2, pair_body, (zero, zero))

    gather_wait(0)
    idx_wait(1)
    gather_start(1)
    sm, gr = compute(0, sm, gr)
    gather_wait(1)
    sm2, gr2 = compute(1, sm, gr)
    valid = (wid + NW * (FULL_ROUNDS + 1)) < TOTAL_CHUNKS
    sm = jnp.where(valid, sm2, sm)
    gr = jnp.where(valid, gr2, gr)

    accs_v[...] = sm
    accg_v[...] = gr
    pltpu.sync_copy(accs_v, out_s.at[wid])
    pltpu.sync_copy(accg_v, out_g.at[wid])


def _combine_tc(pT_ref, tT_ref, ps_ref, pg_ref, out_ref):
    p = pT_ref[...]
    t = tT_ref[...]
    diff = p - t
    recon = jnp.sum(diff * diff) * (1.0 / (3.0 * N))
    th = p[0, :]
    fl = p[1, :]
    ro = p[2, :]
    geo = (
        jnp.sum(jnp.maximum(-th, 0.0))
        + jnp.sum(jnp.maximum(fl - ro + 0.1, 0.0))
        + jnp.sum((th - (ro - fl)) ** 2)
        + jnp.sum(jnp.maximum(th - 20.0, 0.0))
    ) * (1.0 / N)
    smooth = jnp.sum(ps_ref[...]) * (1.0 / (3.0 * E))
    grad = jnp.sum(pg_ref[...]) * (1.0 / (3.0 * E))
    total = recon + LAMBDA_SMOOTH * smooth + LAMBDA_GEO * geo + LAMBDA_GRADIENT * grad
    out_ref[...] = jnp.broadcast_to(total, (1, 1))


def kernel(pred, target, edge_index, positions):
    table = jnp.concatenate(
        [pred, positions[:, :2], jnp.zeros((N, 3), jnp.float32)], axis=1
    )
    part_s, part_g = _edge_loss_sc(table, edge_index)
    out = pl.pallas_call(
        _combine_tc,
        out_shape=jax.ShapeDtypeStruct((1, 1), jnp.float32),
    )(pred.T, target.T, part_s, part_g)
    return out[0, 0]
